# baseline (speedup 1.0000x reference)
import jax
import jax.numpy as jnp
from jax.experimental import pallas as pl
from jax.experimental.pallas import tpu as pltpu
from jax.experimental.pallas import tpu_sc as plsc

_NE = 8192
_D = 64
_NTOK = 16384
_TM = 128
_NBLK = _NTOK // _TM
_TR = _TM * 4
_NC = _NE // 4
_CCOST = 0.25
_GW = 128


def _vq_body(x_ref, embt2_ref,
             dist_ref, enc_ref, idx_ref, loss_ref, perp_ref,
             counts_scr, loss_scr, iota_scr, eid_scr, e2_scr, rep_scr,
             sel_scr):
    b = pl.program_id(0)

    @pl.when(b == 0)
    def _init():
        iota_scr[...] = jax.lax.broadcasted_iota(jnp.int32, (_TM, _NE), 1)
        row4 = jax.lax.broadcasted_iota(jnp.int32, (_TR, _NC), 0)
        col4 = jax.lax.broadcasted_iota(jnp.int32, (_TR, _NC), 1)
        eid_scr[...] = ((row4 % 4) * _NC + col4).astype(jnp.float32)
        embt2 = embt2_ref[...]
        e2_scr[...] = jnp.sum(embt2 * embt2, axis=0, keepdims=True) * 0.25
        rrow = jax.lax.broadcasted_iota(jnp.int32, (_TR, _TM), 0)
        rcol = jax.lax.broadcasted_iota(jnp.int32, (_TR, _TM), 1)
        rep_scr[...] = ((rrow // 4) == rcol).astype(jnp.float32)
        srow = jax.lax.broadcasted_iota(jnp.int32, (4, _TR), 0)
        scol = jax.lax.broadcasted_iota(jnp.int32, (4, _TR), 1)
        sel_scr[...] = ((scol % 4) == srow).astype(jnp.float32)
        counts_scr[...] = jnp.zeros_like(counts_scr)
        loss_scr[0, 0] = 0.0

    x = x_ref[...]

    x2 = jnp.sum(x * x, axis=1, keepdims=True)
    crossm2 = jax.lax.dot_general(x, embt2_ref[...], (((1,), (0,)), ((), ())),
                                  preferred_element_type=jnp.float32)
    dist = (x2 + e2_scr[...]) + crossm2
    dist_ref[...] = dist.reshape(_TR, _NC)

    minval = jnp.min(dist, axis=1, keepdims=True)
    iota = iota_scr[...]
    idx = jnp.min(jnp.where(dist == minval, iota, _NE), axis=1, keepdims=True)
    idx_ref[...] = idx

    idx4 = jax.lax.dot_general(rep_scr[...], idx.astype(jnp.float32),
                               (((1,), (0,)), ((), ())),
                               precision=jax.lax.Precision.HIGHEST,
                               preferred_element_type=jnp.float32)
    enc4 = (eid_scr[...] == idx4).astype(jnp.float32)
    enc_ref[...] = enc4

    counts_scr[...] += jax.lax.dot_general(
        sel_scr[...], enc4, (((1,), (0,)), ((), ())),
        preferred_element_type=jnp.float32)
    loss_scr[0, 0] += jnp.sum(minval)

    @pl.when(b == _NBLK - 1)
    def _fin():
        p = counts_scr[...] * (1.0 / _NTOK)
        ent = -jnp.sum(p * jnp.log(p + 1e-10))
        perp_ref[...] = jnp.reshape(jnp.exp(ent), (1, 1))
        loss_ref[...] = jnp.reshape(
            loss_scr[0, 0] * (_CCOST / (_NTOK * _D)), (1, 1))


def _sc_gather(emb_pad, idx_row):
    mesh = plsc.VectorSubcoreMesh(core_axis_name="c", subcore_axis_name="s")

    @pl.kernel(out_type=jax.ShapeDtypeStruct((_NTOK, 128), jnp.float32),
               mesh=mesh)
    def _k(emb_hbm, i_hbm, o_hbm):
        def body(i_vmem, o_vmem):
            pltpu.sync_copy(emb_hbm.at[i_vmem.at[0]], o_vmem)

        pltpu.emit_pipeline(
            body,
            grid=(_NTOK // _GW,),
            in_specs=[pl.BlockSpec((1, _GW), index_map=lambda i: (0, i))],
            out_specs=[pl.BlockSpec((_GW, 128), index_map=lambda i: (i, 0))],
            core_axis_name="s",
            dimension_semantics=(pltpu.PARALLEL,),
        )(i_hbm, o_hbm)

    return _k(emb_pad, idx_row)


def kernel(inputs, embedding_weight):
    flat_x = jnp.transpose(inputs.reshape(_D, _NTOK), (1, 0))
    embt2 = jnp.transpose(embedding_weight, (1, 0)) * (-2.0)

    out_shapes = (
        jax.ShapeDtypeStruct((_NTOK * 4, _NC), jnp.float32),
        jax.ShapeDtypeStruct((_NTOK * 4, _NC), jnp.float32),
        jax.ShapeDtypeStruct((_NTOK, 1), jnp.int32),
        jax.ShapeDtypeStruct((1, 1), jnp.float32),
        jax.ShapeDtypeStruct((1, 1), jnp.float32),
    )

    dist, enc, idx, loss, perp = pl.pallas_call(
        _vq_body,
        grid=(_NBLK,),
        in_specs=[
            pl.BlockSpec((_TM, _D), lambda b: (b, 0)),
            pl.BlockSpec((_D, _NE), lambda b: (0, 0)),
        ],
        out_specs=(
            pl.BlockSpec((_TR, _NC), lambda b: (b, 0)),
            pl.BlockSpec((_TR, _NC), lambda b: (b, 0)),
            pl.BlockSpec((_TM, 1), lambda b: (b, 0)),
            pl.BlockSpec((1, 1), lambda b: (0, 0)),
            pl.BlockSpec((1, 1), lambda b: (0, 0)),
        ),
        out_shape=out_shapes,
        scratch_shapes=[
            pltpu.VMEM((4, _NC), jnp.float32),
            pltpu.SMEM((1, 1), jnp.float32),
            pltpu.VMEM((_TM, _NE), jnp.int32),
            pltpu.VMEM((_TR, _NC), jnp.float32),
            pltpu.VMEM((1, _NE), jnp.float32),
            pltpu.VMEM((_TR, _TM), jnp.float32),
            pltpu.VMEM((4, _TR), jnp.float32),
        ],
    )(flat_x, embt2)

    emb_pad = jnp.concatenate(
        [embedding_weight,
         jnp.zeros((_NE, 128 - _D), jnp.float32)], axis=1)
    cq = _sc_gather(emb_pad, idx.reshape(1, _NTOK))[:, :_D]
    qst_t = jnp.transpose(cq, (1, 0))

    return (
        loss.reshape(()),
        qst_t.reshape(_D, 16, 1024),
        perp.reshape(()),
        enc.reshape(_D, 1024, _NC),
        dist.reshape(_D, 1024, _NC),
        idx,
        cq,
    )


# speedup vs baseline: 1.0241x; 1.0241x over previous
import jax
import jax.numpy as jnp
from jax.experimental import pallas as pl
from jax.experimental.pallas import tpu as pltpu
from jax.experimental.pallas import tpu_sc as plsc

_NE = 8192
_D = 64
_NTOK = 16384
_TM = 128
_NBLK = _NTOK // _TM
_TR = _TM * 4
_NC = _NE // 4
_CCOST = 0.25
_GW = 128


def _vq_body(x_ref, embt2_ref,
             dist_ref, enc_ref, idx_ref, loss_ref, perp_ref,
             counts_scr, loss_scr, iota_scr, eid_scr, e2_scr, rep_scr,
             idx_scr):
    b = pl.program_id(0)

    @pl.when(b == 0)
    def _init():
        iota_scr[...] = jax.lax.broadcasted_iota(jnp.int32, (_TM, _NE), 1)
        row4 = jax.lax.broadcasted_iota(jnp.int32, (_TR, _NC), 0)
        col4 = jax.lax.broadcasted_iota(jnp.int32, (_TR, _NC), 1)
        eid_scr[...] = ((row4 % 4) * _NC + col4).astype(jnp.float32)
        embt2 = embt2_ref[...]
        e2_scr[...] = jnp.sum(embt2 * embt2, axis=0, keepdims=True) * 0.25
        rrow = jax.lax.broadcasted_iota(jnp.int32, (_TR, _TM), 0)
        rcol = jax.lax.broadcasted_iota(jnp.int32, (_TR, _TM), 1)
        rep_scr[...] = ((rrow // 4) == rcol).astype(jnp.float32)
        counts_scr[...] = jnp.zeros_like(counts_scr)
        loss_scr[...] = jnp.zeros_like(loss_scr)

    x = x_ref[...]

    x2 = jnp.sum(x * x, axis=1, keepdims=True)
    crossm2 = jax.lax.dot_general(x, embt2_ref[...], (((1,), (0,)), ((), ())),
                                  preferred_element_type=jnp.float32)
    dist = (x2 + e2_scr[...]) + crossm2
    dist_ref[...] = dist.reshape(_TR, _NC)

    minval = jnp.min(dist, axis=1, keepdims=True)
    iota = iota_scr[...]
    idx = jnp.min(jnp.where(dist == minval, iota, _NE), axis=1, keepdims=True)
    idx_scr[pl.ds(b * _TM, _TM), :] = idx

    onehot = (iota == idx).astype(jnp.float32)

    idx4 = jax.lax.dot_general(rep_scr[...], idx.astype(jnp.float32),
                               (((1,), (0,)), ((), ())),
                               precision=jax.lax.Precision.HIGHEST,
                               preferred_element_type=jnp.float32)
    enc_ref[...] = (eid_scr[...] == idx4).astype(jnp.float32)

    counts_scr[...] += jnp.sum(onehot, axis=0, keepdims=True)
    loss_scr[...] += minval

    @pl.when(b == _NBLK - 1)
    def _fin():
        p = counts_scr[...] * (1.0 / _NTOK)
        ent = -jnp.sum(p * jnp.log(p + 1e-10))
        perp_ref[...] = jnp.reshape(jnp.exp(ent), (1, 1))
        loss_ref[...] = jnp.reshape(
            jnp.sum(loss_scr[...]) * (_CCOST / (_NTOK * _D)), (1, 1))
        idx_ref[...] = idx_scr[...]


def _sc_gather(emb_pad, idx_row):
    mesh = plsc.VectorSubcoreMesh(core_axis_name="c", subcore_axis_name="s")

    @pl.kernel(out_type=jax.ShapeDtypeStruct((_NTOK, 128), jnp.float32),
               mesh=mesh)
    def _k(emb_hbm, i_hbm, o_hbm):
        def body(i_vmem, o_vmem):
            pltpu.sync_copy(emb_hbm.at[i_vmem.at[0]], o_vmem)

        pltpu.emit_pipeline(
            body,
            grid=(_NTOK // _GW,),
            in_specs=[pl.BlockSpec((1, _GW), index_map=lambda i: (0, i))],
            out_specs=[pl.BlockSpec((_GW, 128), index_map=lambda i: (i, 0))],
            core_axis_name="s",
            dimension_semantics=(pltpu.PARALLEL,),
        )(i_hbm, o_hbm)

    return _k(emb_pad, idx_row)


def kernel(inputs, embedding_weight):
    flat_x = jnp.transpose(inputs.reshape(_D, _NTOK), (1, 0))
    embt2 = jnp.transpose(embedding_weight, (1, 0)) * (-2.0)

    out_shapes = (
        jax.ShapeDtypeStruct((_NTOK * 4, _NC), jnp.float32),
        jax.ShapeDtypeStruct((_NTOK * 4, _NC), jnp.float32),
        jax.ShapeDtypeStruct((_NTOK, 1), jnp.int32),
        jax.ShapeDtypeStruct((1, 1), jnp.float32),
        jax.ShapeDtypeStruct((1, 1), jnp.float32),
    )

    dist, enc, idx, loss, perp = pl.pallas_call(
        _vq_body,
        grid=(_NBLK,),
        in_specs=[
            pl.BlockSpec((_TM, _D), lambda b: (b, 0)),
            pl.BlockSpec((_D, _NE), lambda b: (0, 0)),
        ],
        out_specs=(
            pl.BlockSpec((_TR, _NC), lambda b: (b, 0)),
            pl.BlockSpec((_TR, _NC), lambda b: (b, 0)),
            pl.BlockSpec((_NTOK, 1), lambda b: (0, 0)),
            pl.BlockSpec((1, 1), lambda b: (0, 0)),
            pl.BlockSpec((1, 1), lambda b: (0, 0)),
        ),
        out_shape=out_shapes,
        scratch_shapes=[
            pltpu.VMEM((1, _NE), jnp.float32),
            pltpu.VMEM((_TM, 1), jnp.float32),
            pltpu.VMEM((_TM, _NE), jnp.int32),
            pltpu.VMEM((_TR, _NC), jnp.float32),
            pltpu.VMEM((1, _NE), jnp.float32),
            pltpu.VMEM((_TR, _TM), jnp.float32),
            pltpu.VMEM((_NTOK, 1), jnp.int32),
        ],
    )(flat_x, embt2)

    emb_pad = jnp.concatenate(
        [embedding_weight,
         jnp.zeros((_NE, 128 - _D), jnp.float32)], axis=1)
    cq = _sc_gather(emb_pad, idx.reshape(1, _NTOK))[:, :_D]
    qst_t = jnp.transpose(cq, (1, 0))

    return (
        loss.reshape(()),
        qst_t.reshape(_D, 16, 1024),
        perp.reshape(()),
        enc.reshape(_D, 1024, _NC),
        dist.reshape(_D, 1024, _NC),
        idx,
        cq,
    )


# speedup vs baseline: 1.0377x; 1.0132x over previous
import jax
import jax.numpy as jnp
from jax.experimental import pallas as pl
from jax.experimental.pallas import tpu as pltpu
from jax.experimental.pallas import tpu_sc as plsc

_NE = 8192
_D = 64
_NTOK = 16384
_TM = 128
_NBLK = _NTOK // _TM
_TR = _TM * 4
_NC = _NE // 4
_CCOST = 0.25
_GW = 128


def _vq_body(x_ref, embt2_ref,
             dist_ref, enc_ref, idx_ref, loss_ref, perp_ref,
             counts_scr, loss_scr, iota_scr, eid_scr, e2_scr, rep_scr):
    b = pl.program_id(0)

    @pl.when(b == 0)
    def _init():
        iota_scr[...] = jax.lax.broadcasted_iota(jnp.int32, (_TM, _NE), 1)
        row4 = jax.lax.broadcasted_iota(jnp.int32, (_TR, _NC), 0)
        col4 = jax.lax.broadcasted_iota(jnp.int32, (_TR, _NC), 1)
        eid_scr[...] = ((row4 % 4) * _NC + col4).astype(jnp.float32)
        embt2 = embt2_ref[...]
        e2_scr[...] = jnp.sum(embt2 * embt2, axis=0, keepdims=True) * 0.25
        rrow = jax.lax.broadcasted_iota(jnp.int32, (_TR, _TM), 0)
        rcol = jax.lax.broadcasted_iota(jnp.int32, (_TR, _TM), 1)
        rep_scr[...] = ((rrow // 4) == rcol).astype(jnp.float32)
        counts_scr[...] = jnp.zeros_like(counts_scr)
        loss_scr[0, 0] = 0.0

    x = x_ref[...]

    x2 = jnp.sum(x * x, axis=1, keepdims=True)
    crossm2 = jax.lax.dot_general(x, embt2_ref[...], (((1,), (0,)), ((), ())),
                                  preferred_element_type=jnp.float32)
    dist = (x2 + e2_scr[...]) + crossm2
    dist_ref[...] = dist.reshape(_TR, _NC)

    minval = jnp.min(dist, axis=1, keepdims=True)
    iota = iota_scr[...]
    idx = jnp.min(jnp.where(dist == minval, iota, _NE), axis=1, keepdims=True)
    idx_ref[...] = idx

    onehot = (iota == idx).astype(jnp.float32)

    idx4 = jax.lax.dot_general(rep_scr[...], idx.astype(jnp.float32),
                               (((1,), (0,)), ((), ())),
                               precision=jax.lax.Precision.HIGHEST,
                               preferred_element_type=jnp.float32)
    enc_ref[...] = (eid_scr[...] == idx4).astype(jnp.float32)

    counts_scr[...] += jnp.sum(onehot, axis=0, keepdims=True)
    loss_scr[0, 0] += jnp.sum(minval)

    @pl.when(b == _NBLK - 1)
    def _fin():
        p = counts_scr[...] * (1.0 / _NTOK)
        ent = -jnp.sum(p * jnp.log(p + 1e-10))
        perp_ref[...] = jnp.reshape(jnp.exp(ent), (1, 1))
        loss_ref[...] = jnp.reshape(
            loss_scr[0, 0] * (_CCOST / (_NTOK * _D)), (1, 1))


def _sc_gather(emb_pad, idx_row):
    mesh = plsc.VectorSubcoreMesh(core_axis_name="c", subcore_axis_name="s")

    @pl.kernel(out_type=jax.ShapeDtypeStruct((_NTOK, 128), jnp.float32),
               mesh=mesh)
    def _k(emb_hbm, i_hbm, o_hbm):
        def body(i_vmem, o_vmem):
            pltpu.sync_copy(emb_hbm.at[i_vmem.at[0]], o_vmem)

        pltpu.emit_pipeline(
            body,
            grid=(_NTOK // _GW,),
            in_specs=[pl.BlockSpec((1, _GW), index_map=lambda i: (0, i))],
            out_specs=[pl.BlockSpec((_GW, 128), index_map=lambda i: (i, 0))],
            core_axis_name="s",
            dimension_semantics=(pltpu.PARALLEL,),
        )(i_hbm, o_hbm)

    return _k(emb_pad, idx_row)


def kernel(inputs, embedding_weight):
    flat_x = jnp.transpose(inputs.reshape(_D, _NTOK), (1, 0))
    embt2 = jnp.transpose(embedding_weight, (1, 0)) * (-2.0)

    out_shapes = (
        jax.ShapeDtypeStruct((_NTOK * 4, _NC), jnp.float32),
        jax.ShapeDtypeStruct((_NTOK * 4, _NC), jnp.float32),
        jax.ShapeDtypeStruct((_NTOK, 1), jnp.int32),
        jax.ShapeDtypeStruct((1, 1), jnp.float32),
        jax.ShapeDtypeStruct((1, 1), jnp.float32),
    )

    dist, enc, idx, loss, perp = pl.pallas_call(
        _vq_body,
        grid=(_NBLK,),
        in_specs=[
            pl.BlockSpec((_TM, _D), lambda b: (b, 0)),
            pl.BlockSpec((_D, _NE), lambda b: (0, 0)),
        ],
        out_specs=(
            pl.BlockSpec((_TR, _NC), lambda b: (b, 0)),
            pl.BlockSpec((_TR, _NC), lambda b: (b, 0)),
            pl.BlockSpec((_TM, 1), lambda b: (b, 0)),
            pl.BlockSpec((1, 1), lambda b: (0, 0)),
            pl.BlockSpec((1, 1), lambda b: (0, 0)),
        ),
        out_shape=out_shapes,
        scratch_shapes=[
            pltpu.VMEM((1, _NE), jnp.float32),
            pltpu.SMEM((1, 1), jnp.float32),
            pltpu.VMEM((_TM, _NE), jnp.int32),
            pltpu.VMEM((_TR, _NC), jnp.float32),
            pltpu.VMEM((1, _NE), jnp.float32),
            pltpu.VMEM((_TR, _TM), jnp.float32),
        ],
    )(flat_x, embt2)

    emb_pad = jnp.concatenate(
        [embedding_weight,
         jnp.zeros((_NE, 128 - _D), jnp.float32)], axis=1)
    cq = _sc_gather(emb_pad, idx.reshape(1, _NTOK))[:, :_D]
    qst_t = jnp.transpose(cq, (1, 0))

    return (
        loss.reshape(()),
        qst_t.reshape(_D, 16, 1024),
        perp.reshape(()),
        enc.reshape(_D, 1024, _NC),
        dist.reshape(_D, 1024, _NC),
        idx,
        cq,
    )


# speedup vs baseline: 1.0664x; 1.0277x over previous
import jax
import jax.numpy as jnp
from jax.experimental import pallas as pl
from jax.experimental.pallas import tpu as pltpu
from jax.experimental.pallas import tpu_sc as plsc

_NE = 8192
_D = 64
_NTOK = 16384
_TM = 256
_NBLK = _NTOK // _TM
_TR = _TM * 4
_NC = _NE // 4
_CCOST = 0.25
_GW = 128


def _vq_body(x_ref, embt2_ref,
             dist_ref, enc_ref, idx_ref, loss_ref, perp_ref,
             counts_scr, loss_scr, eid_scr, e2_scr, rep_scr):
    b = pl.program_id(0)

    @pl.when(b == 0)
    def _init():
        row4 = jax.lax.broadcasted_iota(jnp.int32, (_TR, _NC), 0)
        col4 = jax.lax.broadcasted_iota(jnp.int32, (_TR, _NC), 1)
        eid_scr[...] = ((row4 % 4) * _NC + col4).astype(jnp.float32)
        embt2 = embt2_ref[...]
        e2_scr[...] = jnp.sum(embt2 * embt2, axis=0, keepdims=True) * 0.25
        rrow = jax.lax.broadcasted_iota(jnp.int32, (_TR, _TM), 0)
        rcol = jax.lax.broadcasted_iota(jnp.int32, (_TR, _TM), 1)
        rep_scr[...] = ((rrow // 4) == rcol).astype(jnp.float32)
        counts_scr[...] = jnp.zeros_like(counts_scr)
        loss_scr[0, 0] = 0.0

    x = x_ref[...]

    x2 = jnp.sum(x * x, axis=1, keepdims=True)
    crossm2 = jax.lax.dot_general(x, embt2_ref[...], (((1,), (0,)), ((), ())),
                                  preferred_element_type=jnp.float32)
    dist = (x2 + e2_scr[...]) + crossm2
    dist_ref[...] = dist.reshape(_TR, _NC)

    minval = jnp.min(dist, axis=1, keepdims=True)
    iota = jax.lax.broadcasted_iota(jnp.int32, (_TM, _NE), 1)
    idx = jnp.min(jnp.where(dist == minval, iota, _NE), axis=1, keepdims=True)
    idx_ref[...] = idx

    onehot = (iota == idx).astype(jnp.float32)

    idx4 = jax.lax.dot_general(rep_scr[...], idx.astype(jnp.float32),
                               (((1,), (0,)), ((), ())),
                               precision=jax.lax.Precision.HIGHEST,
                               preferred_element_type=jnp.float32)
    enc_ref[...] = (eid_scr[...] == idx4).astype(jnp.float32)

    counts_scr[...] += jnp.sum(onehot, axis=0, keepdims=True)
    loss_scr[0, 0] += jnp.sum(minval)

    @pl.when(b == _NBLK - 1)
    def _fin():
        p = counts_scr[...] * (1.0 / _NTOK)
        ent = -jnp.sum(p * jnp.log(p + 1e-10))
        perp_ref[...] = jnp.reshape(jnp.exp(ent), (1, 1))
        loss_ref[...] = jnp.reshape(
            loss_scr[0, 0] * (_CCOST / (_NTOK * _D)), (1, 1))


def _sc_gather(emb_pad, idx_row):
    mesh = plsc.VectorSubcoreMesh(core_axis_name="c", subcore_axis_name="s")

    @pl.kernel(out_type=jax.ShapeDtypeStruct((_NTOK, 128), jnp.float32),
               mesh=mesh)
    def _k(emb_hbm, i_hbm, o_hbm):
        def body(i_vmem, o_vmem):
            pltpu.sync_copy(emb_hbm.at[i_vmem.at[0]], o_vmem)

        pltpu.emit_pipeline(
            body,
            grid=(_NTOK // _GW,),
            in_specs=[pl.BlockSpec((1, _GW), index_map=lambda i: (0, i))],
            out_specs=[pl.BlockSpec((_GW, 128), index_map=lambda i: (i, 0))],
            core_axis_name="s",
            dimension_semantics=(pltpu.PARALLEL,),
        )(i_hbm, o_hbm)

    return _k(emb_pad, idx_row)


def kernel(inputs, embedding_weight):
    flat_x = jnp.transpose(inputs.reshape(_D, _NTOK), (1, 0))
    embt2 = jnp.transpose(embedding_weight, (1, 0)) * (-2.0)

    out_shapes = (
        jax.ShapeDtypeStruct((_NTOK * 4, _NC), jnp.float32),
        jax.ShapeDtypeStruct((_NTOK * 4, _NC), jnp.float32),
        jax.ShapeDtypeStruct((_NTOK, 1), jnp.int32),
        jax.ShapeDtypeStruct((1, 1), jnp.float32),
        jax.ShapeDtypeStruct((1, 1), jnp.float32),
    )

    dist, enc, idx, loss, perp = pl.pallas_call(
        _vq_body,
        grid=(_NBLK,),
        in_specs=[
            pl.BlockSpec((_TM, _D), lambda b: (b, 0)),
            pl.BlockSpec((_D, _NE), lambda b: (0, 0)),
        ],
        out_specs=(
            pl.BlockSpec((_TR, _NC), lambda b: (b, 0)),
            pl.BlockSpec((_TR, _NC), lambda b: (b, 0)),
            pl.BlockSpec((_TM, 1), lambda b: (b, 0)),
            pl.BlockSpec((1, 1), lambda b: (0, 0)),
            pl.BlockSpec((1, 1), lambda b: (0, 0)),
        ),
        out_shape=out_shapes,
        scratch_shapes=[
            pltpu.VMEM((1, _NE), jnp.float32),
            pltpu.SMEM((1, 1), jnp.float32),
            pltpu.VMEM((_TR, _NC), jnp.float32),
            pltpu.VMEM((1, _NE), jnp.float32),
            pltpu.VMEM((_TR, _TM), jnp.float32),
        ],
    )(flat_x, embt2)

    emb_pad = jnp.concatenate(
        [embedding_weight,
         jnp.zeros((_NE, 128 - _D), jnp.float32)], axis=1)
    cq = _sc_gather(emb_pad, idx.reshape(1, _NTOK))[:, :_D]
    qst_t = jnp.transpose(cq, (1, 0))

    return (
        loss.reshape(()),
        qst_t.reshape(_D, 16, 1024),
        perp.reshape(()),
        enc.reshape(_D, 1024, _NC),
        dist.reshape(_D, 1024, _NC),
        idx,
        cq,
    )
